# Initial kernel scaffold; baseline (speedup 1.0000x reference)
import jax
import jax.numpy as jnp
from jax.experimental import pallas as pl


def kernel(X, nt_emb, W, b):
    raise NotImplementedError("write your pallas kernel here")


# speedup vs baseline: 2.1628x; 2.1628x over previous
import jax
import jax.numpy as jnp
from jax.experimental import pallas as pl

_NT_M = 119
_NP_M = 16
_EMB = 128
_IN = _NT_M + _NP_M
_BLOCK = 2000


def _body(x_ref, table_ref, wpad_ref, b_ref, out_ref):
    x = x_ref[...]
    col = jax.lax.broadcasted_iota(jnp.int32, (1, _IN), 1)
    arange_nt = jnp.where(col < _NT_M, col, 0).astype(jnp.float32)
    idx_f = jnp.sum(x * arange_nt, axis=1, keepdims=True)
    idx = jnp.clip(idx_f.astype(jnp.int32), 0, _NT_M - 1)
    cols = jax.lax.broadcasted_iota(jnp.int32, (x.shape[0], _EMB), 1)
    onehot = (cols == idx).astype(jnp.float32)
    h_nt = jnp.dot(onehot, table_ref[...], preferred_element_type=jnp.float32)
    h_np = jnp.dot(x, wpad_ref[...], preferred_element_type=jnp.float32) + b_ref[...]
    out_ref[...] = jnp.concatenate([h_nt, h_np], axis=1)


def kernel(X, nt_emb, W, b):
    n = X.shape[0]
    table = jnp.zeros((_EMB, _EMB), jnp.float32).at[:_NT_M, :].set(nt_emb)
    wpad = jnp.zeros((_IN, _EMB), jnp.float32).at[_NT_M:, :].set(W.T)
    b2 = b.reshape(1, _EMB)
    grid = (n + _BLOCK - 1) // _BLOCK
    return pl.pallas_call(
        _body,
        grid=(grid,),
        in_specs=[
            pl.BlockSpec((_BLOCK, _IN), lambda i: (i, 0)),
            pl.BlockSpec((_EMB, _EMB), lambda i: (0, 0)),
            pl.BlockSpec((_IN, _EMB), lambda i: (0, 0)),
            pl.BlockSpec((1, _EMB), lambda i: (0, 0)),
        ],
        out_specs=pl.BlockSpec((_BLOCK, 2 * _EMB), lambda i: (i, 0)),
        out_shape=jax.ShapeDtypeStruct((n, 2 * _EMB), jnp.float32),
    )(X, table, wpad, b2)


# speedup vs baseline: 2.3533x; 1.0881x over previous
import jax
import jax.numpy as jnp
from jax.experimental import pallas as pl

_NT_M = 119
_NP_M = 16
_EMB = 128
_IN = _NT_M + _NP_M
_BLOCK = 4000


def _body(x_ref, table_ref, wpad_ref, b_ref, out_ref):
    x = x_ref[...]
    col = jax.lax.broadcasted_iota(jnp.int32, (1, _IN), 1)
    arange_nt = jnp.where(col < _NT_M, col, 0).astype(jnp.float32)
    idx_f = jnp.sum(x * arange_nt, axis=1, keepdims=True)
    idx = jnp.clip(idx_f.astype(jnp.int32), 0, _NT_M - 1)
    cols = jax.lax.broadcasted_iota(jnp.int32, (x.shape[0], _EMB), 1)
    onehot = (cols == idx).astype(jnp.float32)
    h_nt = jnp.dot(onehot, table_ref[...], preferred_element_type=jnp.float32)
    h_np = jnp.dot(x, wpad_ref[...], preferred_element_type=jnp.float32) + b_ref[...]
    out_ref[...] = jnp.concatenate([h_nt, h_np], axis=1)


def kernel(X, nt_emb, W, b):
    n = X.shape[0]
    table = jnp.zeros((_EMB, _EMB), jnp.float32).at[:_NT_M, :].set(nt_emb)
    wpad = jnp.zeros((_IN, _EMB), jnp.float32).at[_NT_M:, :].set(W.T)
    b2 = b.reshape(1, _EMB)
    grid = (n + _BLOCK - 1) // _BLOCK
    return pl.pallas_call(
        _body,
        grid=(grid,),
        in_specs=[
            pl.BlockSpec((_BLOCK, _IN), lambda i: (i, 0)),
            pl.BlockSpec((_EMB, _EMB), lambda i: (0, 0)),
            pl.BlockSpec((_IN, _EMB), lambda i: (0, 0)),
            pl.BlockSpec((1, _EMB), lambda i: (0, 0)),
        ],
        out_specs=pl.BlockSpec((_BLOCK, 2 * _EMB), lambda i: (i, 0)),
        out_shape=jax.ShapeDtypeStruct((n, 2 * _EMB), jnp.float32),
    )(X, table, wpad, b2)


# speedup vs baseline: 2.4017x; 1.0206x over previous
import jax
import jax.numpy as jnp
from jax.experimental import pallas as pl

_NT_M = 119
_NP_M = 16
_EMB = 128
_IN = _NT_M + _NP_M
_BLOCK = 10000


def _body(x_ref, table_ref, wpad_ref, b_ref, out_ref):
    x = x_ref[...]
    col = jax.lax.broadcasted_iota(jnp.int32, (1, _IN), 1)
    arange_nt = jnp.where(col < _NT_M, col, 0).astype(jnp.float32)
    idx_f = jnp.sum(x * arange_nt, axis=1, keepdims=True)
    idx = jnp.clip(idx_f.astype(jnp.int32), 0, _NT_M - 1)
    cols = jax.lax.broadcasted_iota(jnp.int32, (x.shape[0], _EMB), 1)
    onehot = (cols == idx).astype(jnp.float32)
    h_nt = jnp.dot(onehot, table_ref[...], preferred_element_type=jnp.float32)
    h_np = jnp.dot(x, wpad_ref[...], preferred_element_type=jnp.float32) + b_ref[...]
    out_ref[...] = jnp.concatenate([h_nt, h_np], axis=1)


def kernel(X, nt_emb, W, b):
    n = X.shape[0]
    table = jnp.zeros((_EMB, _EMB), jnp.float32).at[:_NT_M, :].set(nt_emb)
    wpad = jnp.zeros((_IN, _EMB), jnp.float32).at[_NT_M:, :].set(W.T)
    b2 = b.reshape(1, _EMB)
    grid = (n + _BLOCK - 1) // _BLOCK
    return pl.pallas_call(
        _body,
        grid=(grid,),
        in_specs=[
            pl.BlockSpec((_BLOCK, _IN), lambda i: (i, 0)),
            pl.BlockSpec((_EMB, _EMB), lambda i: (0, 0)),
            pl.BlockSpec((_IN, _EMB), lambda i: (0, 0)),
            pl.BlockSpec((1, _EMB), lambda i: (0, 0)),
        ],
        out_specs=pl.BlockSpec((_BLOCK, 2 * _EMB), lambda i: (i, 0)),
        out_shape=jax.ShapeDtypeStruct((n, 2 * _EMB), jnp.float32),
    )(X, table, wpad, b2)
